# Initial kernel scaffold; baseline (speedup 1.0000x reference)
import jax
import jax.numpy as jnp
from jax import lax
from jax.experimental import pallas as pl
from jax.experimental.pallas import tpu as pltpu
from jax.experimental.pallas import tpu_sc as plsc

NC = 2
NS = 16
CH = 128


def _lin_norm_body(x_ref, w_ref, b_ref, o_ref):
    h = jnp.dot(x_ref[...], w_ref[...], preferred_element_type=jnp.float32)
    h = h + b_ref[...]
    n = jnp.sqrt(jnp.sum(h * h, axis=1, keepdims=True))
    o_ref[...] = h / jnp.maximum(n, 1e-12)


def _combine_lin_norm_body(p_ref, c_ref, w_ref, b_ref, o_ref):
    agg = p_ref[0] + p_ref[1]
    cnt = c_ref[0][:, 0:1] + c_ref[1][:, 0:1]
    agg = agg / jnp.maximum(cnt, 1.0)
    h = jnp.maximum(agg, 0.0)
    h = jnp.dot(h, w_ref[...], preferred_element_type=jnp.float32) + b_ref[...]
    n = jnp.sqrt(jnp.sum(h * h, axis=1, keepdims=True))
    o_ref[...] = h / jnp.maximum(n, 1e-12)


def _post_body(q_ref, c_ref, w1_ref, b1_ref, w2_ref, b2_ref, o_ref):
    agg = q_ref[0] + q_ref[1]
    cnt = c_ref[0][:, 0:1] + c_ref[1][:, 0:1]
    agg = agg / jnp.maximum(cnt, 1.0)
    h = jnp.maximum(agg, 0.0)
    t = jnp.dot(h, w1_ref[...], preferred_element_type=jnp.float32) + b1_ref[...]
    o = jnp.dot(t, w2_ref[...], preferred_element_type=jnp.float32) + b2_ref[...]
    m = jnp.max(o, axis=1, keepdims=True)
    e = jnp.exp(o - m)
    s = jnp.sum(e, axis=1, keepdims=True)
    o_ref[...] = o - m - jnp.log(s)


def _tc_lin_norm(xp, W, b, NP, BR):
    grid = NP // BR
    return pl.pallas_call(
        _lin_norm_body,
        grid=(grid,),
        in_specs=[
            pl.BlockSpec((BR, 128), lambda i: (i, 0)),
            pl.BlockSpec((128, 128), lambda i: (0, 0)),
            pl.BlockSpec((1, 128), lambda i: (0, 0)),
        ],
        out_specs=pl.BlockSpec((BR, 128), lambda i: (i, 0)),
        out_shape=jax.ShapeDtypeStruct((NP, 128), jnp.float32),
    )(xp, W, b.reshape(1, 128))


def _tc_combine_lin_norm(p, c, W, b, NP, BR):
    grid = NP // BR
    return pl.pallas_call(
        _combine_lin_norm_body,
        grid=(grid,),
        in_specs=[
            pl.BlockSpec((NC, BR, 128), lambda i: (0, i, 0)),
            pl.BlockSpec((NC, BR, 16), lambda i: (0, i, 0)),
            pl.BlockSpec((128, 128), lambda i: (0, 0)),
            pl.BlockSpec((1, 128), lambda i: (0, 0)),
        ],
        out_specs=pl.BlockSpec((BR, 128), lambda i: (i, 0)),
        out_shape=jax.ShapeDtypeStruct((NP, 128), jnp.float32),
    )(p, c, W, b.reshape(1, 128))


def _tc_post(q, c, W1, b1, W2, b2, NP, BR, DOUT):
    grid = NP // BR
    return pl.pallas_call(
        _post_body,
        grid=(grid,),
        in_specs=[
            pl.BlockSpec((NC, BR, 128), lambda i: (0, i, 0)),
            pl.BlockSpec((NC, BR, 16), lambda i: (0, i, 0)),
            pl.BlockSpec((128, 128), lambda i: (0, 0)),
            pl.BlockSpec((1, 128), lambda i: (0, 0)),
            pl.BlockSpec((128, DOUT), lambda i: (0, 0)),
            pl.BlockSpec((1, DOUT), lambda i: (0, 0)),
        ],
        out_specs=pl.BlockSpec((BR, DOUT), lambda i: (i, 0)),
        out_shape=jax.ShapeDtypeStruct((NP, DOUT), jnp.float32),
    )(q, c, W1, b1.reshape(1, 128), W2, b2.reshape(1, DOUT))


def _make_sc_agg(NP, K, with_counts):
    mesh = plsc.VectorSubcoreMesh(core_axis_name="c", subcore_axis_name="s")
    out_type = [jax.ShapeDtypeStruct((NC, NP, 128), jnp.float32)]
    scratch = [
        pltpu.VMEM((K, CH), jnp.int32),
        pltpu.VMEM((K, CH), jnp.int32),
        pltpu.VMEM((CH, 128), jnp.float32),
        pltpu.VMEM_SHARED((NP, 128), jnp.float32),
        pltpu.SemaphoreType.DMA,
    ]
    if with_counts:
        out_type.append(jax.ShapeDtypeStruct((NC, NP, 16), jnp.float32))
        scratch += [
            pltpu.VMEM((CH, 16), jnp.float32),
            pltpu.VMEM_SHARED((NP, 16), jnp.float32),
        ]
    RPT = NP // NS

    def body(*refs):
        if with_counts:
            (h_hbm, src_hbm, dst_hbm, z_hbm, zc_hbm, ones_hbm,
             agg_out, cnt_out,
             src_v, dst_v, rows_v, accum_sh, sem, ones_v, cnt_sh) = refs
        else:
            (h_hbm, src_hbm, dst_hbm, z_hbm,
             agg_out,
             src_v, dst_v, rows_v, accum_sh, sem) = refs
        cid = lax.axis_index("c")
        sid = lax.axis_index("s")
        r0 = sid * RPT
        pltpu.sync_copy(z_hbm.at[pl.ds(r0, RPT)], accum_sh.at[pl.ds(r0, RPT)])
        pltpu.sync_copy(src_hbm.at[cid, sid], src_v)
        pltpu.sync_copy(dst_hbm.at[cid, sid], dst_v)
        if with_counts:
            pltpu.sync_copy(zc_hbm.at[pl.ds(r0, RPT)], cnt_sh.at[pl.ds(r0, RPT)])
            pltpu.sync_copy(ones_hbm, ones_v)
        plsc.subcore_barrier()

        def step(j, carry):
            pltpu.async_copy(h_hbm.at[src_v.at[j]], rows_v, sem).wait()
            pltpu.sync_copy(rows_v, accum_sh.at[dst_v.at[j]], add=True)
            if with_counts:
                pltpu.sync_copy(ones_v, cnt_sh.at[dst_v.at[j]], add=True)
            return carry

        lax.fori_loop(0, K, step, 0)
        plsc.subcore_barrier()
        pltpu.sync_copy(accum_sh.at[pl.ds(r0, RPT)],
                        agg_out.at[cid, pl.ds(r0, RPT)])
        if with_counts:
            pltpu.sync_copy(cnt_sh.at[pl.ds(r0, RPT)],
                            cnt_out.at[cid, pl.ds(r0, RPT)])

    return pl.kernel(body, out_type=out_type, mesh=mesh, scratch_types=scratch)


def kernel(x, edge_index, batch, W0, b0, W1, b1, Wp1, bp1, Wp2, bp2):
    N, D = x.shape
    DOUT = Wp2.shape[1]
    E = edge_index.shape[1]
    BR = 1280
    NP = ((N + 16 + BR - 1) // BR) * BR
    K = -(-E // (NC * NS * CH))
    EP = NC * NS * K * CH

    src = edge_index[0].astype(jnp.int32)
    dst = edge_index[1].astype(jnp.int32)
    src_p = jnp.concatenate([src, jnp.zeros((EP - E,), jnp.int32)])
    dst_p = jnp.concatenate([dst, jnp.full((EP - E,), N, jnp.int32)])
    src4 = src_p.reshape(NC, NS, K, CH)
    dst4 = dst_p.reshape(NC, NS, K, CH)

    xp = jnp.pad(x, ((0, NP - N), (0, 0)))
    z128 = jnp.zeros((NP, 128), jnp.float32)
    z16 = jnp.zeros((NP, 16), jnp.float32)
    ones16 = jnp.zeros((CH, 16), jnp.float32).at[:, 0].set(1.0)

    sc_agg_cnt = _make_sc_agg(NP, K, with_counts=True)
    sc_agg = _make_sc_agg(NP, K, with_counts=False)

    h0 = _tc_lin_norm(xp, W0, b0, NP, BR)
    p0, c0 = sc_agg_cnt(h0, src4, dst4, z128, z16, ones16)
    h1 = _tc_combine_lin_norm(p0, c0, W1, b1, NP, BR)
    p1 = sc_agg(h1, src4, dst4, z128)
    out = _tc_post(p1, c0, Wp1, bp1, Wp2, bp2, NP, BR, DOUT)
    return out[:N]


# speedup vs baseline: 5.1483x; 5.1483x over previous
import jax
import jax.numpy as jnp
from jax import lax
from jax.experimental import pallas as pl
from jax.experimental.pallas import tpu as pltpu
from jax.experimental.pallas import tpu_sc as plsc

NC = 2
NS = 16
CH = 128


def _lin_norm_body(x_ref, w_ref, b_ref, o_ref):
    h = jnp.dot(x_ref[...], w_ref[...], preferred_element_type=jnp.float32)
    h = h + b_ref[...]
    n = jnp.sqrt(jnp.sum(h * h, axis=1, keepdims=True))
    o_ref[...] = h / jnp.maximum(n, 1e-12)


def _combine_lin_norm_body(p_ref, c_ref, w_ref, b_ref, o_ref):
    agg = p_ref[0] + p_ref[1]
    cnt = c_ref[0][:, 0:1] + c_ref[1][:, 0:1]
    agg = agg / jnp.maximum(cnt, 1.0)
    h = jnp.maximum(agg, 0.0)
    h = jnp.dot(h, w_ref[...], preferred_element_type=jnp.float32) + b_ref[...]
    n = jnp.sqrt(jnp.sum(h * h, axis=1, keepdims=True))
    o_ref[...] = h / jnp.maximum(n, 1e-12)


def _post_body(q_ref, c_ref, w1_ref, b1_ref, w2_ref, b2_ref, o_ref):
    agg = q_ref[0] + q_ref[1]
    cnt = c_ref[0][:, 0:1] + c_ref[1][:, 0:1]
    agg = agg / jnp.maximum(cnt, 1.0)
    h = jnp.maximum(agg, 0.0)
    t = jnp.dot(h, w1_ref[...], preferred_element_type=jnp.float32) + b1_ref[...]
    o = jnp.dot(t, w2_ref[...], preferred_element_type=jnp.float32) + b2_ref[...]
    m = jnp.max(o, axis=1, keepdims=True)
    e = jnp.exp(o - m)
    s = jnp.sum(e, axis=1, keepdims=True)
    o_ref[...] = o - m - jnp.log(s)


def _tc_lin_norm(xp, W, b, NP, BR):
    grid = NP // BR
    return pl.pallas_call(
        _lin_norm_body,
        grid=(grid,),
        in_specs=[
            pl.BlockSpec((BR, 128), lambda i: (i, 0)),
            pl.BlockSpec((128, 128), lambda i: (0, 0)),
            pl.BlockSpec((1, 128), lambda i: (0, 0)),
        ],
        out_specs=pl.BlockSpec((BR, 128), lambda i: (i, 0)),
        out_shape=jax.ShapeDtypeStruct((NP, 128), jnp.float32),
    )(xp, W, b.reshape(1, 128))


def _tc_combine_lin_norm(p, c, W, b, NP, BR):
    grid = NP // BR
    return pl.pallas_call(
        _combine_lin_norm_body,
        grid=(grid,),
        in_specs=[
            pl.BlockSpec((NC, BR, 128), lambda i: (0, i, 0)),
            pl.BlockSpec((NC, BR, 128), lambda i: (0, i, 0)),
            pl.BlockSpec((128, 128), lambda i: (0, 0)),
            pl.BlockSpec((1, 128), lambda i: (0, 0)),
        ],
        out_specs=pl.BlockSpec((BR, 128), lambda i: (i, 0)),
        out_shape=jax.ShapeDtypeStruct((NP, 128), jnp.float32),
    )(p, c, W, b.reshape(1, 128))


def _tc_post(q, c, W1, b1, W2, b2, NP, BR, DOUT):
    grid = NP // BR
    return pl.pallas_call(
        _post_body,
        grid=(grid,),
        in_specs=[
            pl.BlockSpec((NC, BR, 128), lambda i: (0, i, 0)),
            pl.BlockSpec((NC, BR, 128), lambda i: (0, i, 0)),
            pl.BlockSpec((128, 128), lambda i: (0, 0)),
            pl.BlockSpec((1, 128), lambda i: (0, 0)),
            pl.BlockSpec((128, DOUT), lambda i: (0, 0)),
            pl.BlockSpec((1, DOUT), lambda i: (0, 0)),
        ],
        out_specs=pl.BlockSpec((BR, DOUT), lambda i: (i, 0)),
        out_shape=jax.ShapeDtypeStruct((NP, DOUT), jnp.float32),
    )(q, c, W1, b1.reshape(1, 128), W2, b2.reshape(1, DOUT))


def _make_sc_agg(NP, NSP, K):
    mesh = plsc.VectorSubcoreMesh(core_axis_name="c", subcore_axis_name="s")
    out_type = jax.ShapeDtypeStruct((NC, NP, 128), jnp.float32)
    scratch = [
        pltpu.VMEM((K, CH), jnp.int32),
        pltpu.VMEM((K, CH), jnp.int32),
        pltpu.VMEM((CH, 128), jnp.float32),
        pltpu.VMEM_SHARED((NSP, 128), jnp.float32),
        pltpu.SemaphoreType.DMA,
    ]
    RPT = NSP // NS

    def body(h_hbm, src_hbm, dst_hbm, z_hbm, agg_out,
             src_v, dst_v, rows_v, accum_sh, sem):
        cid = lax.axis_index("c")
        sid = lax.axis_index("s")
        r0 = sid * RPT
        pltpu.sync_copy(z_hbm.at[pl.ds(r0, RPT)], accum_sh.at[pl.ds(r0, RPT)])
        pltpu.sync_copy(src_hbm.at[cid, sid], src_v)
        pltpu.sync_copy(dst_hbm.at[cid, sid], dst_v)
        plsc.subcore_barrier()

        def step(j, carry):
            pltpu.async_copy(h_hbm.at[src_v.at[j]], rows_v, sem).wait()
            pltpu.sync_copy(rows_v, accum_sh.at[dst_v.at[j]], add=True)
            return carry

        lax.fori_loop(0, K, step, 0)
        plsc.subcore_barrier()
        pltpu.sync_copy(accum_sh.at[pl.ds(r0, RPT)],
                        agg_out.at[cid, pl.ds(r0, RPT)])

    return pl.kernel(body, out_type=out_type, mesh=mesh, scratch_types=scratch)


def _make_sc_counts(NP, NSP, K):
    mesh = plsc.VectorSubcoreMesh(core_axis_name="c", subcore_axis_name="s")
    out_type = jax.ShapeDtypeStruct((NC, NP, 128), jnp.float32)
    RPT = NSP // NS
    scratch = [
        pltpu.VMEM((K, CH), jnp.int32),
        pltpu.VMEM((CH, 128), jnp.float32),
        pltpu.VMEM_SHARED((NSP, 128), jnp.float32),
    ]

    def body(dst_hbm, z_hbm, ones_hbm, cnt_out, dst_v, ones_v, cnt_sh):
        cid = lax.axis_index("c")
        sid = lax.axis_index("s")
        r0 = sid * RPT
        pltpu.sync_copy(z_hbm.at[pl.ds(r0, RPT)], cnt_sh.at[pl.ds(r0, RPT)])
        pltpu.sync_copy(dst_hbm.at[cid, sid], dst_v)
        pltpu.sync_copy(ones_hbm, ones_v)
        plsc.subcore_barrier()

        def step(j, carry):
            pltpu.sync_copy(ones_v, cnt_sh.at[dst_v.at[j]], add=True)
            return carry

        lax.fori_loop(0, K, step, 0)
        plsc.subcore_barrier()
        pltpu.sync_copy(cnt_sh.at[pl.ds(r0, RPT)],
                        cnt_out.at[cid, pl.ds(r0, RPT)])

    return pl.kernel(body, out_type=out_type, mesh=mesh, scratch_types=scratch)


def kernel(x, edge_index, batch, W0, b0, W1, b1, Wp1, bp1, Wp2, bp2):
    N, D = x.shape
    DOUT = Wp2.shape[1]
    E = edge_index.shape[1]
    BR = 1280
    NP = ((N + 16 + BR - 1) // BR) * BR
    NSP = ((N + 16 + 8 * NS - 1) // (8 * NS)) * (8 * NS)
    K = -(-E // (NC * NS * CH))
    EP = NC * NS * K * CH

    src = edge_index[0].astype(jnp.int32)
    dst = edge_index[1].astype(jnp.int32)
    src_p = jnp.concatenate([src, jnp.zeros((EP - E,), jnp.int32)])
    dst_p = jnp.concatenate([dst, jnp.full((EP - E,), N, jnp.int32)])
    src4 = src_p.reshape(NC, NS, K, CH)
    dst4 = dst_p.reshape(NC, NS, K, CH)

    xp = jnp.pad(x, ((0, NP - N), (0, 0)))
    z128 = jnp.zeros((NP, 128), jnp.float32)
    ones128 = jnp.zeros((CH, 128), jnp.float32).at[:, 0].set(1.0)

    sc_agg = _make_sc_agg(NP, NSP, K)
    sc_counts = _make_sc_counts(NP, NSP, K)

    h0 = _tc_lin_norm(xp, W0, b0, NP, BR)
    c0 = sc_counts(dst4, z128, ones128)
    p0 = sc_agg(h0, src4, dst4, z128)
    h1 = _tc_combine_lin_norm(p0, c0, W1, b1, NP, BR)
    p1 = sc_agg(h1, src4, dst4, z128)
    out = _tc_post(p1, c0, Wp1, bp1, Wp2, bp2, NP, BR, DOUT)
    return out[:N]
